# baseline (speedup 1.0000x reference)
import functools

import jax
import jax.numpy as jnp
from jax import lax
from jax.experimental import pallas as pl
from jax.experimental.pallas import tpu as pltpu
from jax.experimental.pallas import tpu_sc as plsc

L = 16
NC, NS = 2, 16
NW = NC * NS


def kernel(objs, vectors, lengths):
    B, = objs.shape
    N, P, D = vectors.shape

    bpw = B // NW
    ngrp = bpw // L
    LCH = 128

    mesh = plsc.VectorSubcoreMesh(core_axis_name="c", subcore_axis_name="s",
                                  num_cores=NC, num_subcores=NS)

    @functools.partial(
        pl.kernel,
        out_type=jax.ShapeDtypeStruct((B, D), jnp.float32),
        mesh=mesh,
        compiler_params=pltpu.CompilerParams(use_tc_tiling_on_sc=True),
        scratch_types=[
            pltpu.VMEM((bpw,), jnp.int32),
            pltpu.VMEM((bpw,), jnp.int32),
            pltpu.VMEM((bpw + L,), jnp.float32),
            pltpu.VMEM((L, P, D), jnp.float32),
            pltpu.VMEM((L, P, D), jnp.float32),
            pltpu.VMEM((bpw, D), jnp.float32),
            pltpu.SemaphoreType.DMA,
            pltpu.SemaphoreType.DMA,
            pltpu.SemaphoreType.DMA,
        ],
    )
    def sc_kernel(objs_hbm, vec_hbm, len_hbm, out_hbm,
                  idx_v, lens_v, recip_v, slabs0, slabs1, out_v,
                  lsem, sem0, sem1):
        wid = lax.axis_index("s") * NC + lax.axis_index("c")
        base = wid * bpw
        slabs = (slabs0, slabs1)
        sems = (sem0, sem1)

        pltpu.sync_copy(objs_hbm.at[pl.ds(base, bpw)], idx_v)

        lens_handles = [
            pltpu.async_copy(len_hbm.at[idx_v.at[pl.ds(k * LCH, LCH)]],
                             lens_v.at[pl.ds(k * LCH, LCH)], lsem)
            for k in range(bpw // LCH)
        ]

        def fire_group(g, b):
            v16 = idx_v[pl.ds(g * L, L)]
            for k in range(L):
                s = lax.squeeze(lax.slice(v16, (k,), (k + 1,)), (0,))
                pltpu.async_copy(vec_hbm.at[s], slabs[b].at[k], sems[b])

        fire_group(0, 0)
        fire_group(1, 1)

        for h in lens_handles:
            h.wait()
        for g in range(ngrp):
            lv = lens_v[pl.ds(g * L, L)]
            recip_v[pl.ds(g * L, L)] = 1.0 / (lv.astype(jnp.float32) + 1e-5)

        zero_idx = jnp.zeros((L, 1), jnp.int32)
        bcast_dnums = lax.GatherDimensionNumbers(
            offset_dims=(), collapsed_slice_dims=(0,), start_index_map=(0,))

        def bcast0(v):
            return lax.gather(v, zero_idx, bcast_dnums, (1,),
                              mode=lax.GatherScatterMode.PROMISE_IN_BOUNDS)

        def body(i, _):
            for b in range(2):
                g = 2 * i + b
                pltpu.make_async_copy(
                    vec_hbm.at[pl.ds(0, L)], slabs[b], sems[b]).wait()
                for k in range(L):
                    j = g * L + k
                    rcp = bcast0(recip_v[pl.ds(j, L)])
                    for db in range(D // L):
                        acc = slabs[b][k, 0, pl.ds(db * L, L)]
                        for p in range(1, P):
                            acc = acc + slabs[b][k, p, pl.ds(db * L, L)]
                        out_v[j, pl.ds(db * L, L)] = acc * rcp

                @pl.when(g + 2 < ngrp)
                def _fire(g=g, b=b):
                    fire_group(g + 2, b)
            return 0

        lax.fori_loop(0, ngrp // 2, body, 0)

        pltpu.sync_copy(out_v, out_hbm.at[pl.ds(base, bpw)])

    return sc_kernel(objs, vectors, lengths)
